# baseline (speedup 1.0000x reference)
import jax
import jax.numpy as jnp
from jax import lax
from jax.experimental import pallas as pl
from jax.experimental.pallas import tpu as pltpu
from jax.experimental.pallas import tpu_sc as plsc

_TOPK = 8
_B = 256
_K = 17
_HW = 96 * 72
_PSC = 2304
_PTC = _HW - _PSC
_NC = 1
_NS = 16
_L = 16
_CB = _B // _NC
_NG = _CB // _L
_PPT = _PSC // _NS
_NSLOT = 3
_CP = _PPT // _NSLOT
_PUNROLL = 4
_NSTEP = _CP // _PUNROLL
_RB = 1152
_JTC = _PTC // _RB
_JOFF = _PSC // _RB


def _sc_body(x_hbm, y_hbm, out_hbm, o_buf, t_buf, partial_v, sems):
    c = lax.axis_index("c")
    s = lax.axis_index("s")
    cbase = c * _CB
    pbase = s * _PPT
    z = jnp.zeros((_L,), jnp.float32)

    def o_copy(ki, slot):
        return pltpu.make_async_copy(
            x_hbm.at[ki, pl.ds(pbase + slot * _CP, _CP), pl.ds(cbase, _CB)],
            o_buf.at[slot], sems.at[slot])

    def t_copy(ki, slot):
        return pltpu.make_async_copy(
            y_hbm.at[ki, pl.ds(pbase + slot * _CP, _CP), pl.ds(cbase, _CB)],
            t_buf.at[slot], sems.at[_NSLOT + slot])

    for slot in range(_NSLOT):
        o_copy(0, slot).start()
        t_copy(0, slot).start()

    def outer(ki, carry):
        accs = (z,) * _NG
        for slot in range(_NSLOT):
            o_copy(ki, slot).wait()
            t_copy(ki, slot).wait()

            def inner(i, a, _slot=slot):
                a = list(a)
                for u in range(_PUNROLL):
                    pp = i * _PUNROLL + u
                    for gg in range(_NG):
                        ov = o_buf[_slot, pp, pl.ds(gg * _L, _L)]
                        tv = t_buf[_slot, pp, pl.ds(gg * _L, _L)]
                        d = ov - tv
                        a[gg] = a[gg] + d * d
                return tuple(a)

            accs = lax.fori_loop(0, _NSTEP, inner, accs)

            @pl.when(ki + 1 < _K)
            def _():
                o_copy(ki + 1, slot).start()
                t_copy(ki + 1, slot).start()

        for gg in range(_NG):
            partial_v[ki, pl.ds(gg * _L, _L)] = accs[gg]
        return carry

    lax.fori_loop(0, _K, outer, 0)
    pltpu.sync_copy(partial_v, out_hbm.at[c, s])


def _tc_stage1(x_ref, y_ref, o_ref):
    j = pl.program_id(1)
    d = x_ref[0] - y_ref[0]
    part = jnp.sum(d * d, axis=0, keepdims=True)

    @pl.when(j == 0)
    def _():
        o_ref[0] = part

    @pl.when(j > 0)
    def _():
        o_ref[0] = o_ref[0] + part


def _tc_finish(p_ref, q_ref, w_ref, o_ref):
    x = p_ref[...]
    losses = (jnp.sum(x, axis=1) + q_ref[...]) * w_ref[...]
    tot = jnp.zeros((_NC, 1, _CB), jnp.float32)
    for _ in range(_TOPK):
        m = jnp.max(losses, axis=1, keepdims=True)
        tot = tot + m
        eq = losses == m
        taken = jnp.zeros((_NC, 1, _CB), jnp.bool_)
        cols = []
        for k in range(_K):
            ek = eq[:, k:k + 1, :] & (~taken)
            cols.append(jnp.where(ek, -1.0, losses[:, k:k + 1, :]))
            taken = taken | ek
        losses = jnp.concatenate(cols, axis=1)
    o_ref[...] = jnp.full((1, _CB), jnp.sum(tot) * (1.0 / (_TOPK * _B)),
                          jnp.float32)


def kernel(output, target, target_weight):
    x3 = output.transpose(1, 2, 3, 0).reshape(_K, _HW, _B)
    y3 = target.transpose(1, 2, 3, 0).reshape(_K, _HW, _B)
    wsq = ((target_weight * target_weight).reshape(_B, _K).T * (1.0 / _HW))
    wsq = wsq.reshape(_K, _NC, _CB).transpose(1, 0, 2)

    mesh = plsc.VectorSubcoreMesh(core_axis_name="c", subcore_axis_name="s", num_cores=1)
    sc = pl.kernel(
        _sc_body,
        out_type=jax.ShapeDtypeStruct((_NC, _NS, _K, _CB), jnp.float32),
        mesh=mesh,
        compiler_params=pltpu.CompilerParams(
            needs_layout_passes=False, skip_device_barrier=True),
        scratch_types=[
            pltpu.VMEM((_NSLOT, _CP, _CB), jnp.float32),
            pltpu.VMEM((_NSLOT, _CP, _CB), jnp.float32),
            pltpu.VMEM((_K, _CB), jnp.float32),
            pltpu.SemaphoreType.DMA((2 * _NSLOT,)),
        ],
    )
    sc_partials = sc(x3, y3)

    tc_losses = pl.pallas_call(
        _tc_stage1,
        grid=(_K, _JTC),
        in_specs=[
            pl.BlockSpec((1, _RB, _B), lambda k, j: (k, j + _JOFF, 0)),
            pl.BlockSpec((1, _RB, _B), lambda k, j: (k, j + _JOFF, 0)),
        ],
        out_specs=pl.BlockSpec((1, 1, _B), lambda k, j: (k, 0, 0)),
        out_shape=jax.ShapeDtypeStruct((_K, 1, _B), jnp.float32),
    )(x3, y3)
    tc_part = tc_losses.reshape(_K, _NC, _CB).transpose(1, 0, 2)

    total = pl.pallas_call(
        _tc_finish,
        out_shape=jax.ShapeDtypeStruct((1, _CB), jnp.float32),
    )(sc_partials, tc_part, wsq)
    return total[0, 0]


# speedup vs baseline: 1.4570x; 1.4570x over previous
import jax
import jax.numpy as jnp
from jax import lax
from jax.experimental import pallas as pl
from jax.experimental.pallas import tpu as pltpu
from jax.experimental.pallas import tpu_sc as plsc

_TOPK = 8
_B = 256
_K = 17
_HW = 96 * 72
_PSC = 3456
_PTC = _HW - _PSC
_NC = 2
_NS = 16
_L = 16
_CB = _B // _NC
_NG = _CB // _L
_PPT = _PSC // _NS
_NSLOT = 3
_CP = _PPT // _NSLOT
_PUNROLL = 4
_NSTEP = _CP // _PUNROLL
_RB = 1728
_JTC = _PTC // _RB
_JOFF = _PSC // _RB


def _sc_body(x_hbm, y_hbm, out_hbm, o_buf, t_buf, partial_v, sems):
    c = lax.axis_index("c")
    s = lax.axis_index("s")
    cbase = c * _CB
    pbase = s * _PPT
    z = jnp.zeros((_L,), jnp.float32)

    def o_copy(ki, slot):
        return pltpu.make_async_copy(
            x_hbm.at[ki, pl.ds(pbase + slot * _CP, _CP), pl.ds(cbase, _CB)],
            o_buf.at[slot], sems.at[slot])

    def t_copy(ki, slot):
        return pltpu.make_async_copy(
            y_hbm.at[ki, pl.ds(pbase + slot * _CP, _CP), pl.ds(cbase, _CB)],
            t_buf.at[slot], sems.at[_NSLOT + slot])

    for slot in range(_NSLOT):
        o_copy(0, slot).start()
        t_copy(0, slot).start()

    def outer(ki, carry):
        accs = (z,) * _NG
        for slot in range(_NSLOT):
            o_copy(ki, slot).wait()
            t_copy(ki, slot).wait()

            def inner(i, a, _slot=slot):
                a = list(a)
                for u in range(_PUNROLL):
                    pp = i * _PUNROLL + u
                    for gg in range(_NG):
                        ov = o_buf[_slot, pp, pl.ds(gg * _L, _L)]
                        tv = t_buf[_slot, pp, pl.ds(gg * _L, _L)]
                        d = ov - tv
                        a[gg] = a[gg] + d * d
                return tuple(a)

            accs = lax.fori_loop(0, _NSTEP, inner, accs)

            @pl.when(ki + 1 < _K)
            def _():
                o_copy(ki + 1, slot).start()
                t_copy(ki + 1, slot).start()

        for gg in range(_NG):
            partial_v[ki, pl.ds(gg * _L, _L)] = accs[gg]
        return carry

    lax.fori_loop(0, _K, outer, 0)
    pltpu.sync_copy(partial_v, out_hbm.at[c, s])


def _tc_stage1(x_ref, y_ref, o_ref):
    j = pl.program_id(1)
    d = x_ref[0] - y_ref[0]
    part = jnp.sum(d * d, axis=0, keepdims=True)

    @pl.when(j == 0)
    def _():
        o_ref[0] = part

    @pl.when(j > 0)
    def _():
        o_ref[0] = o_ref[0] + part


def _tc_finish(p_ref, q_ref, w_ref, o_ref):
    x = p_ref[...]
    losses = (jnp.sum(x, axis=1) + q_ref[...]) * w_ref[...]
    tot = jnp.zeros((_NC, 1, _CB), jnp.float32)
    for _ in range(_TOPK):
        m = jnp.max(losses, axis=1, keepdims=True)
        tot = tot + m
        eq = losses == m
        taken = jnp.zeros((_NC, 1, _CB), jnp.bool_)
        cols = []
        for k in range(_K):
            ek = eq[:, k:k + 1, :] & (~taken)
            cols.append(jnp.where(ek, -1.0, losses[:, k:k + 1, :]))
            taken = taken | ek
        losses = jnp.concatenate(cols, axis=1)
    o_ref[...] = jnp.full((1, _CB), jnp.sum(tot) * (1.0 / (_TOPK * _B)),
                          jnp.float32)


def kernel(output, target, target_weight):
    x3 = output.transpose(1, 2, 3, 0).reshape(_K, _HW, _B)
    y3 = target.transpose(1, 2, 3, 0).reshape(_K, _HW, _B)
    wsq = ((target_weight * target_weight).reshape(_B, _K).T * (1.0 / _HW))
    wsq = wsq.reshape(_K, _NC, _CB).transpose(1, 0, 2)

    mesh = plsc.VectorSubcoreMesh(core_axis_name="c", subcore_axis_name="s")
    sc = pl.kernel(
        _sc_body,
        out_type=jax.ShapeDtypeStruct((_NC, _NS, _K, _CB), jnp.float32),
        mesh=mesh,
        compiler_params=pltpu.CompilerParams(
            needs_layout_passes=False, skip_device_barrier=True),
        scratch_types=[
            pltpu.VMEM((_NSLOT, _CP, _CB), jnp.float32),
            pltpu.VMEM((_NSLOT, _CP, _CB), jnp.float32),
            pltpu.VMEM((_K, _CB), jnp.float32),
            pltpu.SemaphoreType.DMA((2 * _NSLOT,)),
        ],
    )
    sc_partials = sc(x3, y3)

    tc_losses = pl.pallas_call(
        _tc_stage1,
        grid=(_K, _JTC),
        in_specs=[
            pl.BlockSpec((1, _RB, _B), lambda k, j: (k, j + _JOFF, 0)),
            pl.BlockSpec((1, _RB, _B), lambda k, j: (k, j + _JOFF, 0)),
        ],
        out_specs=pl.BlockSpec((1, 1, _B), lambda k, j: (k, 0, 0)),
        out_shape=jax.ShapeDtypeStruct((_K, 1, _B), jnp.float32),
    )(x3, y3)
    tc_part = tc_losses.reshape(_K, _NC, _CB).transpose(1, 0, 2)

    total = pl.pallas_call(
        _tc_finish,
        out_shape=jax.ShapeDtypeStruct((1, _CB), jnp.float32),
    )(sc_partials, tc_part, wsq)
    return total[0, 0]
